# baseline (speedup 1.0000x reference)
import jax
import jax.numpy as jnp
from jax.experimental import pallas as pl
from jax.experimental.pallas import tpu as pltpu


def _support_body(x1_ref, x2_ref, w_ref, s_ref):
    f = w_ref.shape[1]
    s_ref[:, :f] = jnp.dot(x1_ref[...], w_ref[...],
                           preferred_element_type=jnp.float32)
    s_ref[:, f:] = jnp.dot(x2_ref[...], w_ref[...],
                           preferred_element_type=jnp.float32)


def _pick_block(n, target):
    for bm in range(min(target, n), 7, -1):
        if n % bm == 0 and bm % 8 == 0:
            return bm
    return n


def kernel(feat, feat_a, fadj, W_z, W_x):
    n, fin = feat.shape
    fhid = W_z.shape[1]
    fout = W_x.shape[1]

    bm = _pick_block(n, 400)
    bc = 2560 if n > 2560 else n
    nbr = n // bm
    nbc = -(-n // bc)
    bc_edge = n - (nbc - 1) * bc

    bs = _pick_block(n, 2000)
    s1 = pl.pallas_call(
        _support_body,
        grid=(n // bs,),
        in_specs=[
            pl.BlockSpec((bs, fin), lambda i: (i, 0)),
            pl.BlockSpec((bs, fin), lambda i: (i, 0)),
            pl.BlockSpec((fin, fhid), lambda i: (0, 0)),
        ],
        out_specs=pl.BlockSpec((bs, 2 * fhid), lambda i: (i, 0)),
        out_shape=jax.ShapeDtypeStruct((n, 2 * fhid), jnp.float32),
        compiler_params=pltpu.CompilerParams(
            dimension_semantics=("arbitrary",)),
    )(feat, feat_a, W_z)

    def _fused_body(a_ref, s1_ref, wx_ref,
                    z1_ref, z2_ref, xh1_ref, xh2_ref,
                    s2_ref, zacc_ref, xacc_ref):
        phase = pl.program_id(0)
        i = pl.program_id(1)
        j = pl.program_id(2)
        f = fout

        @pl.when((phase == 0) & (i == 0) & (j == 0))
        def _init():
            xacc_ref[...] = jnp.zeros_like(xacc_ref)

        done0 = ((j + 1) * bc) <= (i * bm)

        @pl.when((phase == 0) & (j < nbc - 1))
        def _p0_full():
            a = a_ref[...]
            mac = jnp.dot(a, s1_ref[pl.ds(j * bc, bc), :],
                          preferred_element_type=jnp.float32)

            @pl.when(j == 0)
            def _():
                zacc_ref[...] = mac

            @pl.when(j > 0)
            def _():
                zacc_ref[...] = zacc_ref[...] + mac

            @pl.when(done0)
            def _():
                xacc_ref[pl.ds(i * bm, bm), :] = (
                    xacc_ref[pl.ds(i * bm, bm), :]
                    + jnp.dot(a, s2_ref[pl.ds(j * bc, bc), :],
                              preferred_element_type=jnp.float32))

        @pl.when((phase == 0) & (j == nbc - 1))
        def _p0_edge():
            mac = jnp.dot(a_ref[:, :bc_edge],
                          s1_ref[pl.ds((nbc - 1) * bc, bc_edge), :],
                          preferred_element_type=jnp.float32)
            zfull = mac if nbc == 1 else zacc_ref[...] + mac
            z1 = zfull[:, :f]
            z2 = zfull[:, f:]
            z1_ref[...] = z1
            z2_ref[...] = z2
            s2_ref[pl.ds(i * bm, bm), :f] = jnp.dot(
                z1, wx_ref[...], preferred_element_type=jnp.float32)
            s2_ref[pl.ds(i * bm, bm), f:] = jnp.dot(
                z2, wx_ref[...], preferred_element_type=jnp.float32)

        @pl.when((phase == 1) & (j < nbc - 1) & jnp.logical_not(done0))
        def _p1_full():
            xacc_ref[pl.ds(i * bm, bm), :] = (
                xacc_ref[pl.ds(i * bm, bm), :]
                + jnp.dot(a_ref[...], s2_ref[pl.ds(j * bc, bc), :],
                          preferred_element_type=jnp.float32))

        @pl.when((phase == 1) & (j == nbc - 1))
        def _p1_edge():
            mac = jnp.dot(a_ref[:, :bc_edge],
                          s2_ref[pl.ds((nbc - 1) * bc, bc_edge), :],
                          preferred_element_type=jnp.float32)
            acc = mac if nbc == 1 else xacc_ref[pl.ds(i * bm, bm), :] + mac
            xh1_ref[...] = acc[:, :f]
            xh2_ref[...] = acc[:, f:]

    def _a_index(l, i, j):
        jmin = (i * bm) // bc
        return (i, jnp.where(l == 0, j, jnp.maximum(j, jmin)))

    res = pl.pallas_call(
        _fused_body,
        grid=(2, nbr, nbc),
        in_specs=[
            pl.BlockSpec((bm, bc), _a_index),
            pl.BlockSpec((n, 2 * fhid), lambda l, i, j: (0, 0)),
            pl.BlockSpec((fhid, fout), lambda l, i, j: (0, 0)),
        ],
        out_specs=[
            pl.BlockSpec((bm, fhid),
                         lambda l, i, j: ((1 - l) * i + l * (nbr - 1), 0)),
            pl.BlockSpec((bm, fhid),
                         lambda l, i, j: ((1 - l) * i + l * (nbr - 1), 0)),
            pl.BlockSpec((bm, fout), lambda l, i, j: (l * i, 0)),
            pl.BlockSpec((bm, fout), lambda l, i, j: (l * i, 0)),
        ],
        out_shape=[
            jax.ShapeDtypeStruct((n, fhid), jnp.float32),
            jax.ShapeDtypeStruct((n, fhid), jnp.float32),
            jax.ShapeDtypeStruct((n, fout), jnp.float32),
            jax.ShapeDtypeStruct((n, fout), jnp.float32),
        ],
        scratch_shapes=[
            pltpu.VMEM((n, 2 * fout), jnp.float32),
            pltpu.VMEM((bm, 2 * fhid), jnp.float32),
            pltpu.VMEM((n, 2 * fout), jnp.float32),
        ],
        compiler_params=pltpu.CompilerParams(
            dimension_semantics=("arbitrary", "arbitrary", "arbitrary")),
    )(fadj, s1, W_x)

    z_ori, z_aug, xhat_ori, xhat_aug = res
    return (z_ori, z_aug, xhat_ori, xhat_aug)


# speedup vs baseline: 1.0353x; 1.0353x over previous
import numpy as np

import jax
import jax.numpy as jnp
from jax.experimental import pallas as pl
from jax.experimental.pallas import tpu as pltpu


def _support_body(x1_ref, x2_ref, w_ref, s_ref):
    f = w_ref.shape[1]
    s_ref[:, :f] = jnp.dot(x1_ref[...], w_ref[...],
                           preferred_element_type=jnp.float32)
    s_ref[:, f:] = jnp.dot(x2_ref[...], w_ref[...],
                           preferred_element_type=jnp.float32)


def _pick_block(n, target):
    for bm in range(min(target, n), 7, -1):
        if n % bm == 0 and bm % 8 == 0:
            return bm
    return n


def kernel(feat, feat_a, fadj, W_z, W_x):
    n, fin = feat.shape
    fhid = W_z.shape[1]
    fout = W_x.shape[1]

    bm = _pick_block(n, 400)
    bc = 2560 if n > 2560 else n
    nbr = n // bm
    nbc = -(-n // bc)
    bc_edge = n - (nbc - 1) * bc

    bs = _pick_block(n, 2000)
    s1 = pl.pallas_call(
        _support_body,
        grid=(n // bs,),
        in_specs=[
            pl.BlockSpec((bs, fin), lambda i: (i, 0)),
            pl.BlockSpec((bs, fin), lambda i: (i, 0)),
            pl.BlockSpec((fin, fhid), lambda i: (0, 0)),
        ],
        out_specs=pl.BlockSpec((bs, 2 * fhid), lambda i: (i, 0)),
        out_shape=jax.ShapeDtypeStruct((n, 2 * fhid), jnp.float32),
        compiler_params=pltpu.CompilerParams(
            dimension_semantics=("arbitrary",)),
    )(feat, feat_a, W_z)

    steps = []
    for i in range(nbr):
        for j in range(nbc):
            steps.append((0, i, j))
    for i in range(nbr):
        jmin = (i * bm) // bc
        for j in range(jmin, nbc):
            steps.append((1, i, j))
    steps_np = np.asarray(steps, dtype=np.int32)
    ph_np = jnp.asarray(steps_np[:, 0])
    i_np = jnp.asarray(steps_np[:, 1])
    j_np = jnp.asarray(steps_np[:, 2])
    num_steps = len(steps)

    def _fused_body(ph_ref, i_ref, j_ref, a_ref, s1_ref, wx_ref,
                    z1_ref, z2_ref, xh1_ref, xh2_ref,
                    s2_ref, zacc_ref, xacc_ref):
        t = pl.program_id(0)
        phase = ph_ref[t]
        i = i_ref[t]
        j = j_ref[t]
        f = fout

        @pl.when(t == 0)
        def _init():
            xacc_ref[...] = jnp.zeros_like(xacc_ref)

        done0 = ((j + 1) * bc) <= (i * bm)

        @pl.when((phase == 0) & (j < nbc - 1))
        def _p0_full():
            a = a_ref[...]
            mac = jnp.dot(a, s1_ref[pl.ds(j * bc, bc), :],
                          preferred_element_type=jnp.float32)

            @pl.when(j == 0)
            def _():
                zacc_ref[...] = mac

            @pl.when(j > 0)
            def _():
                zacc_ref[...] = zacc_ref[...] + mac

            @pl.when(done0)
            def _():
                xacc_ref[pl.ds(i * bm, bm), :] = (
                    xacc_ref[pl.ds(i * bm, bm), :]
                    + jnp.dot(a, s2_ref[pl.ds(j * bc, bc), :],
                              preferred_element_type=jnp.float32))

        @pl.when((phase == 0) & (j == nbc - 1))
        def _p0_edge():
            mac = jnp.dot(a_ref[:, :bc_edge],
                          s1_ref[pl.ds((nbc - 1) * bc, bc_edge), :],
                          preferred_element_type=jnp.float32)
            zfull = mac if nbc == 1 else zacc_ref[...] + mac
            z1 = zfull[:, :f]
            z2 = zfull[:, f:]
            z1_ref[...] = z1
            z2_ref[...] = z2
            s2_ref[pl.ds(i * bm, bm), :f] = jnp.dot(
                z1, wx_ref[...], preferred_element_type=jnp.float32)
            s2_ref[pl.ds(i * bm, bm), f:] = jnp.dot(
                z2, wx_ref[...], preferred_element_type=jnp.float32)

        @pl.when((phase == 1) & (j < nbc - 1))
        def _p1_full():
            xacc_ref[pl.ds(i * bm, bm), :] = (
                xacc_ref[pl.ds(i * bm, bm), :]
                + jnp.dot(a_ref[...], s2_ref[pl.ds(j * bc, bc), :],
                          preferred_element_type=jnp.float32))

        @pl.when((phase == 1) & (j == nbc - 1))
        def _p1_edge():
            mac = jnp.dot(a_ref[:, :bc_edge],
                          s2_ref[pl.ds((nbc - 1) * bc, bc_edge), :],
                          preferred_element_type=jnp.float32)
            acc = mac if nbc == 1 else xacc_ref[pl.ds(i * bm, bm), :] + mac
            xh1_ref[...] = acc[:, :f]
            xh2_ref[...] = acc[:, f:]

    grid_spec = pltpu.PrefetchScalarGridSpec(
        num_scalar_prefetch=3,
        grid=(num_steps,),
        in_specs=[
            pl.BlockSpec((bm, bc),
                         lambda t, ph, ii, jj: (ii[t], jj[t])),
            pl.BlockSpec((n, 2 * fhid), lambda t, ph, ii, jj: (0, 0)),
            pl.BlockSpec((fhid, fout), lambda t, ph, ii, jj: (0, 0)),
        ],
        out_specs=[
            pl.BlockSpec((bm, fhid),
                         lambda t, ph, ii, jj: (
                             jnp.where(ph[t] == 0, ii[t], nbr - 1), 0)),
            pl.BlockSpec((bm, fhid),
                         lambda t, ph, ii, jj: (
                             jnp.where(ph[t] == 0, ii[t], nbr - 1), 0)),
            pl.BlockSpec((bm, fout),
                         lambda t, ph, ii, jj: (
                             jnp.where(ph[t] == 1, ii[t], 0), 0)),
            pl.BlockSpec((bm, fout),
                         lambda t, ph, ii, jj: (
                             jnp.where(ph[t] == 1, ii[t], 0), 0)),
        ],
        scratch_shapes=[
            pltpu.VMEM((n, 2 * fout), jnp.float32),
            pltpu.VMEM((bm, 2 * fhid), jnp.float32),
            pltpu.VMEM((n, 2 * fout), jnp.float32),
        ],
    )

    res = pl.pallas_call(
        _fused_body,
        grid_spec=grid_spec,
        out_shape=[
            jax.ShapeDtypeStruct((n, fhid), jnp.float32),
            jax.ShapeDtypeStruct((n, fhid), jnp.float32),
            jax.ShapeDtypeStruct((n, fout), jnp.float32),
            jax.ShapeDtypeStruct((n, fout), jnp.float32),
        ],
        compiler_params=pltpu.CompilerParams(
            dimension_semantics=("arbitrary",)),
    )(ph_np, i_np, j_np, fadj, s1, W_x)

    z_ori, z_aug, xhat_ori, xhat_aug = res
    return (z_ori, z_aug, xhat_ori, xhat_aug)


# speedup vs baseline: 1.2073x; 1.1661x over previous
import numpy as np

import jax
import jax.numpy as jnp
from jax.experimental import pallas as pl
from jax.experimental.pallas import tpu as pltpu


def _support_body(x1_ref, x2_ref, w_ref, s_ref):
    f = w_ref.shape[1]
    s_ref[:, :f] = jnp.dot(x1_ref[...], w_ref[...],
                           preferred_element_type=jnp.float32)
    s_ref[:, f:] = jnp.dot(x2_ref[...], w_ref[...],
                           preferred_element_type=jnp.float32)


def _pick_block(n, target):
    for bm in range(min(target, n), 7, -1):
        if n % bm == 0 and bm % 8 == 0:
            return bm
    return n


def kernel(feat, feat_a, fadj, W_z, W_x):
    n, fin = feat.shape
    fhid = W_z.shape[1]
    fout = W_x.shape[1]

    bm = _pick_block(n, 400)
    bc = 5120 if n > 5120 else n
    nbr = n // bm
    nbc = -(-n // bc)
    bc_edge = n - (nbc - 1) * bc

    bs = _pick_block(n, 2000)
    s1 = pl.pallas_call(
        _support_body,
        grid=(n // bs,),
        in_specs=[
            pl.BlockSpec((bs, fin), lambda i: (i, 0)),
            pl.BlockSpec((bs, fin), lambda i: (i, 0)),
            pl.BlockSpec((fin, fhid), lambda i: (0, 0)),
        ],
        out_specs=pl.BlockSpec((bs, 2 * fhid), lambda i: (i, 0)),
        out_shape=jax.ShapeDtypeStruct((n, 2 * fhid), jnp.float32),
        compiler_params=pltpu.CompilerParams(
            dimension_semantics=("arbitrary",)),
    )(feat, feat_a, W_z)

    steps = []
    for i in range(nbr):
        for j in range(nbc):
            steps.append((0, i, j))
    for i in range(nbr):
        jmin = (i * bm) // bc
        for j in range(jmin, nbc):
            steps.append((1, i, j))
    steps_np = np.asarray(steps, dtype=np.int32)
    ph_np = jnp.asarray(steps_np[:, 0])
    i_np = jnp.asarray(steps_np[:, 1])
    j_np = jnp.asarray(steps_np[:, 2])
    num_steps = len(steps)

    def _fused_body(ph_ref, i_ref, j_ref, a_ref, s1_ref, wx_ref,
                    z1_ref, z2_ref, xh1_ref, xh2_ref,
                    s2_ref, zacc_ref, xacc_ref):
        t = pl.program_id(0)
        phase = ph_ref[t]
        i = i_ref[t]
        j = j_ref[t]
        f = fout

        @pl.when(t == 0)
        def _init():
            xacc_ref[...] = jnp.zeros_like(xacc_ref)

        done0 = ((j + 1) * bc) <= (i * bm)

        @pl.when((phase == 0) & (j < nbc - 1))
        def _p0_full():
            a = a_ref[...]
            mac = jnp.dot(a, s1_ref[pl.ds(j * bc, bc), :],
                          preferred_element_type=jnp.float32)

            @pl.when(j == 0)
            def _():
                zacc_ref[...] = mac

            @pl.when(j > 0)
            def _():
                zacc_ref[...] = zacc_ref[...] + mac

            @pl.when(done0)
            def _():
                xacc_ref[pl.ds(i * bm, bm), :] = (
                    xacc_ref[pl.ds(i * bm, bm), :]
                    + jnp.dot(a, s2_ref[pl.ds(j * bc, bc), :],
                              preferred_element_type=jnp.float32))

        @pl.when((phase == 0) & (j == nbc - 1))
        def _p0_edge():
            mac = jnp.dot(a_ref[:, :bc_edge],
                          s1_ref[pl.ds((nbc - 1) * bc, bc_edge), :],
                          preferred_element_type=jnp.float32)
            zfull = mac if nbc == 1 else zacc_ref[...] + mac
            z1 = zfull[:, :f]
            z2 = zfull[:, f:]
            z1_ref[...] = z1
            z2_ref[...] = z2
            s2_ref[pl.ds(i * bm, bm), :f] = jnp.dot(
                z1, wx_ref[...], preferred_element_type=jnp.float32)
            s2_ref[pl.ds(i * bm, bm), f:] = jnp.dot(
                z2, wx_ref[...], preferred_element_type=jnp.float32)

        @pl.when((phase == 1) & (j < nbc - 1))
        def _p1_full():
            xacc_ref[pl.ds(i * bm, bm), :] = (
                xacc_ref[pl.ds(i * bm, bm), :]
                + jnp.dot(a_ref[...], s2_ref[pl.ds(j * bc, bc), :],
                          preferred_element_type=jnp.float32))

        @pl.when((phase == 1) & (j == nbc - 1))
        def _p1_edge():
            mac = jnp.dot(a_ref[:, :bc_edge],
                          s2_ref[pl.ds((nbc - 1) * bc, bc_edge), :],
                          preferred_element_type=jnp.float32)
            acc = mac if nbc == 1 else xacc_ref[pl.ds(i * bm, bm), :] + mac
            xh1_ref[...] = acc[:, :f]
            xh2_ref[...] = acc[:, f:]

    grid_spec = pltpu.PrefetchScalarGridSpec(
        num_scalar_prefetch=3,
        grid=(num_steps,),
        in_specs=[
            pl.BlockSpec((bm, bc),
                         lambda t, ph, ii, jj: (ii[t], jj[t])),
            pl.BlockSpec((n, 2 * fhid), lambda t, ph, ii, jj: (0, 0)),
            pl.BlockSpec((fhid, fout), lambda t, ph, ii, jj: (0, 0)),
        ],
        out_specs=[
            pl.BlockSpec((bm, fhid),
                         lambda t, ph, ii, jj: (
                             jnp.where(ph[t] == 0, ii[t], nbr - 1), 0)),
            pl.BlockSpec((bm, fhid),
                         lambda t, ph, ii, jj: (
                             jnp.where(ph[t] == 0, ii[t], nbr - 1), 0)),
            pl.BlockSpec((bm, fout),
                         lambda t, ph, ii, jj: (
                             jnp.where(ph[t] == 1, ii[t], 0), 0)),
            pl.BlockSpec((bm, fout),
                         lambda t, ph, ii, jj: (
                             jnp.where(ph[t] == 1, ii[t], 0), 0)),
        ],
        scratch_shapes=[
            pltpu.VMEM((n, 2 * fout), jnp.float32),
            pltpu.VMEM((bm, 2 * fhid), jnp.float32),
            pltpu.VMEM((n, 2 * fout), jnp.float32),
        ],
    )

    res = pl.pallas_call(
        _fused_body,
        grid_spec=grid_spec,
        out_shape=[
            jax.ShapeDtypeStruct((n, fhid), jnp.float32),
            jax.ShapeDtypeStruct((n, fhid), jnp.float32),
            jax.ShapeDtypeStruct((n, fout), jnp.float32),
            jax.ShapeDtypeStruct((n, fout), jnp.float32),
        ],
        compiler_params=pltpu.CompilerParams(
            dimension_semantics=("arbitrary",)),
    )(ph_np, i_np, j_np, fadj, s1, W_x)

    z_ori, z_aug, xhat_ori, xhat_aug = res
    return (z_ori, z_aug, xhat_ori, xhat_aug)
